# Initial kernel scaffold; baseline (speedup 1.0000x reference)
import jax
import jax.numpy as jnp
from jax.experimental import pallas as pl


def kernel(indices, table):
    raise NotImplementedError("write your pallas kernel here")


# speedup vs baseline: 1.8462x; 1.8462x over previous
import functools

import jax
import jax.numpy as jnp
from jax import lax
from jax.experimental import pallas as pl
from jax.experimental.pallas import tpu as pltpu
from jax.experimental.pallas import tpu_sc as plsc

_NC = 2
_NS = 16
_NW = _NC * _NS
_SUB = 128
_K = 8


def kernel(indices, table):
    nb, nh = indices.shape
    B = nb * nh
    D = table.shape[1]
    idx2d = indices.reshape(B // _SUB, _SUB).astype(jnp.int32)

    b_per_w = B // _NW
    rows_per_w = b_per_w // _SUB
    n_it = rows_per_w // _K

    mesh = plsc.VectorSubcoreMesh(core_axis_name="c", subcore_axis_name="s")

    @functools.partial(
        pl.kernel,
        mesh=mesh,
        out_type=jax.ShapeDtypeStruct((B, D), jnp.float32),
        scratch_types=[
            pltpu.VMEM((_K, _SUB), jnp.int32),
            pltpu.VMEM((_K * _SUB, D), jnp.float32),
            pltpu.SemaphoreType.DMA,
        ],
        compiler_params=pltpu.CompilerParams(use_tc_tiling_on_sc=False),
    )
    def gather_k(table_hbm, idx_hbm, out_hbm, idx_v, rows_v, sem):
        wid = lax.axis_index("s") * _NC + lax.axis_index("c")
        idx_row0 = wid * rows_per_w
        out_row0 = wid * b_per_w

        def step(i, carry):
            pltpu.sync_copy(idx_hbm.at[pl.ds(idx_row0 + i * _K, _K)], idx_v)
            copies = [
                pltpu.async_copy(
                    table_hbm.at[idx_v.at[j]],
                    rows_v.at[pl.ds(j * _SUB, _SUB)],
                    sem,
                )
                for j in range(_K)
            ]
            for c in copies:
                c.wait()
            pltpu.sync_copy(
                rows_v, out_hbm.at[pl.ds(out_row0 + i * (_K * _SUB), _K * _SUB)]
            )
            return carry

        lax.fori_loop(0, n_it, step, 0)

    out = gather_k(table, idx2d)
    return out.reshape(nb, nh, D)


# speedup vs baseline: 1.8733x; 1.0147x over previous
import functools

import jax
import jax.numpy as jnp
from jax import lax
from jax.experimental import pallas as pl
from jax.experimental.pallas import tpu as pltpu
from jax.experimental.pallas import tpu_sc as plsc

_NC = 2
_NS = 16
_NW = _NC * _NS
_SUB = 128
_K = 5
_CH = _K * _SUB


def kernel(indices, table):
    nb, nh = indices.shape
    B = nb * nh
    D = table.shape[1]
    idx_flat = indices.reshape(B).astype(jnp.int32)

    b_per_w = B // _NW
    n_it = b_per_w // _CH
    n_pairs = n_it // 2

    mesh = plsc.VectorSubcoreMesh(core_axis_name="c", subcore_axis_name="s")

    @functools.partial(
        pl.kernel,
        mesh=mesh,
        out_type=jax.ShapeDtypeStruct((B, D), jnp.float32),
        scratch_types=[
            pltpu.VMEM((2 * _CH,), jnp.int32),
            pltpu.VMEM((_CH, D), jnp.float32),
            pltpu.VMEM((_CH, D), jnp.float32),
            pltpu.SemaphoreType.DMA,
            pltpu.SemaphoreType.DMA,
            pltpu.SemaphoreType.DMA,
            pltpu.SemaphoreType.DMA,
        ],
        compiler_params=pltpu.CompilerParams(use_tc_tiling_on_sc=False),
    )
    def gather_k(table_hbm, idx_hbm, out_hbm, idx_v, rows0, rows1,
                 g0, g1, o0, o1):
        wid = lax.axis_index("s") * _NC + lax.axis_index("c")
        base = wid * b_per_w
        rows = (rows0, rows1)
        sem_g = (g0, g1)
        sem_o = (o0, o1)

        def fire(ci, b):
            pltpu.sync_copy(idx_hbm.at[pl.ds(base + ci * _CH, _CH)],
                            idx_v.at[pl.ds(b * _CH, _CH)])
            for j in range(_K):
                pltpu.async_copy(
                    table_hbm.at[idx_v.at[pl.ds(b * _CH + j * _SUB, _SUB)]],
                    rows[b].at[pl.ds(j * _SUB, _SUB)],
                    sem_g[b],
                )

        def drain_gathers(b):
            for j in range(_K):
                pltpu.make_async_copy(
                    table_hbm.at[idx_v.at[pl.ds(b * _CH + j * _SUB, _SUB)]],
                    rows[b].at[pl.ds(j * _SUB, _SUB)],
                    sem_g[b],
                ).wait()

        def start_writeback(ci, b):
            pltpu.async_copy(
                rows[b], out_hbm.at[pl.ds(base + ci * _CH, _CH)], sem_o[b]
            )

        def wait_writeback(b):
            pltpu.make_async_copy(
                rows[b], out_hbm.at[pl.ds(base, _CH)], sem_o[b]
            ).wait()

        fire(0, 0)
        fire(1, 1)

        def pair(g, carry):
            ci = 2 * g
            drain_gathers(0)
            start_writeback(ci, 0)
            drain_gathers(1)
            start_writeback(ci + 1, 1)
            wait_writeback(0)
            fire(ci + 2, 0)
            wait_writeback(1)
            fire(ci + 3, 1)
            return carry

        lax.fori_loop(0, n_pairs - 1, pair, 0)

        ci = 2 * (n_pairs - 1)
        drain_gathers(0)
        start_writeback(ci, 0)
        drain_gathers(1)
        start_writeback(ci + 1, 1)
        wait_writeback(0)
        wait_writeback(1)

    out = gather_k(table, idx_flat)
    return out.reshape(nb, nh, D)
